# baseline (speedup 1.0000x reference)
import functools

import jax
import jax.numpy as jnp
from jax import lax
from jax.experimental import pallas as pl
from jax.experimental.pallas import tpu as pltpu
from jax.experimental.pallas import tpu_sc as plsc

_D = 64
_SCALE = 8.0
_NW = 32
_C = 256
_LANES = 16


def _build(n_total):
    per_w = n_total // _NW
    n_chunks = per_w // _C
    mesh = plsc.VectorSubcoreMesh(core_axis_name="c", subcore_axis_name="s")

    @functools.partial(
        pl.kernel,
        mesh=mesh,
        out_type=jax.ShapeDtypeStruct((n_total, _D), jnp.float32),
        scratch_types=[
            pltpu.VMEM((per_w,), jnp.int32),
            pltpu.VMEM((_C, _D), jnp.float32),
            pltpu.VMEM((_C, _D), jnp.float32),
            pltpu.VMEM((_C, _D), jnp.float32),
            pltpu.SemaphoreType.DMA,
            pltpu.SemaphoreType.DMA,
            pltpu.SemaphoreType.DMA,
            pltpu.SemaphoreType.DMA,
            pltpu.SemaphoreType.DMA,
            pltpu.SemaphoreType.DMA,
        ],
    )
    def k(x_hbm, table_hbm, out_hbm, idx_v, rb0, rb1, rb2,
          g0, g1, g2, o0, o1, o2):
        rbs = (rb0, rb1, rb2)
        gsems = (g0, g1, g2)
        osems = (o0, o1, o2)
        wid = lax.axis_index("s") * 2 + lax.axis_index("c")
        base = wid * per_w
        pltpu.sync_copy(x_hbm.at[pl.ds(base, per_w)], idx_v)

        def enqueue(j, rb, gsem):
            def grp(g, carry):
                v = idx_v[pl.ds(j * _C + g * _LANES, _LANES)]
                for l in range(_LANES):
                    s = v[l]
                    pltpu.async_copy(
                        table_hbm.at[pl.ds(s, 1)],
                        rb.at[pl.ds(g * _LANES + l, 1)],
                        gsem,
                    )
                return carry

            lax.fori_loop(0, _C // _LANES, grp, 0)

        def drain(rb, gsem):
            pltpu.make_async_copy(table_hbm.at[pl.ds(0, _C)], rb, gsem).wait()

        def scale(rb):
            def row(r, carry):
                for c in range(_D // _LANES):
                    sl = pl.ds(c * _LANES, _LANES)
                    rb[r, sl] = rb[r, sl] * _SCALE
                return carry

            lax.fori_loop(0, _C, row, 0, unroll=4)

        def put(j, rb, osem):
            pltpu.async_copy(rb, out_hbm.at[pl.ds(base + j * _C, _C)], osem)

        def wait_put(rb, osem):
            pltpu.make_async_copy(rb, out_hbm.at[pl.ds(0, _C)], osem).wait()

        enqueue(0, rbs[0], gsems[0])

        def body(j, carry):
            def go(b):
                nb = (b + 1) % 3
                rb, gsem, osem = rbs[b], gsems[b], osems[b]
                nrb, ngsem, nosem = rbs[nb], gsems[nb], osems[nb]

                @pl.when(j + 1 < n_chunks)
                def _():
                    @pl.when(j >= 2)
                    def _():
                        wait_put(nrb, nosem)

                    enqueue(j + 1, nrb, ngsem)

                drain(rb, gsem)
                scale(rb)
                put(j, rb, osem)

            lax.switch(lax.rem(j, 3), [lambda: go(0), lambda: go(1), lambda: go(2)])
            return carry

        lax.fori_loop(0, n_chunks, body, 0)
        wait_put(rbs[(n_chunks - 3) % 3], osems[(n_chunks - 3) % 3])
        wait_put(rbs[(n_chunks - 2) % 3], osems[(n_chunks - 2) % 3])
        wait_put(rbs[(n_chunks - 1) % 3], osems[(n_chunks - 1) % 3])

    return k


def kernel(x, table):
    b, t = x.shape
    n_total = b * t
    xf = x.reshape(n_total)
    out = _build(n_total)(xf, table)
    return out.reshape(b, t, _D)
